# baseline (speedup 1.0000x reference)
import functools

import jax
import jax.numpy as jnp
from jax import lax
from jax.experimental import pallas as pl
from jax.experimental.pallas import tpu as pltpu
from jax.experimental.pallas import tpu_sc as plsc


def kernel(idx, token_embedding_table, pos_encoding):
    B, T = idx.shape
    V, D = token_embedding_table.shape
    G = T // 2
    NSEG = 2

    info = plsc.get_sparse_core_info()
    NC, NS = info.num_cores, info.num_subcores
    NW = NC * NS
    n_chunks = B
    chunks_per_w = n_chunks // NW

    idx2 = idx.reshape(B * NSEG, G)
    pe2 = pos_encoding.reshape(NSEG, G, D)

    mesh = plsc.VectorSubcoreMesh(core_axis_name="c", subcore_axis_name="s")

    @functools.partial(
        pl.kernel,
        mesh=mesh,
        compiler_params=pltpu.CompilerParams(use_tc_tiling_on_sc=False),
        out_type=jax.ShapeDtypeStruct((n_chunks, NSEG, G, D), jnp.float32),
        scratch_types=[
            pltpu.VMEM((NSEG, G), jnp.int32),
            pltpu.VMEM((NSEG, G, D), jnp.float32),
            pltpu.VMEM((NSEG, G, D), jnp.float32),
            pltpu.SemaphoreType.DMA,
        ],
    )
    def run(idx_hbm, table_hbm, pe_hbm, out_hbm, idx_v, rows_v, pe_v, sem):
        wid = lax.axis_index("s") * NC + lax.axis_index("c")
        pltpu.sync_copy(pe_hbm, pe_v)

        def chunk_body(t, carry):
            c = wid * chunks_per_w + t
            pltpu.sync_copy(idx_hbm.at[pl.ds(c * NSEG, NSEG)], idx_v)
            for j in range(NSEG):
                pltpu.async_copy(table_hbm.at[idx_v.at[j]], rows_v.at[j], sem).wait()

            def add_body(r, carry2):
                for j in range(NSEG):
                    for u in range(D // 16):
                        sl = pl.ds(u * 16, 16)
                        rows_v[j, r, sl] = rows_v[j, r, sl] + pe_v[j, r, sl]
                return carry2

            lax.fori_loop(0, G, add_body, 0)
            pltpu.sync_copy(rows_v, out_hbm.at[c])
            return carry

        lax.fori_loop(0, chunks_per_w, chunk_body, 0)

    out = run(idx2, token_embedding_table, pe2)
    return out.reshape(B, T, D)


# speedup vs baseline: 1.3016x; 1.3016x over previous
import functools

import jax
import jax.numpy as jnp
from jax import lax
from jax.experimental import pallas as pl
from jax.experimental.pallas import tpu as pltpu
from jax.experimental.pallas import tpu_sc as plsc


def kernel(idx, token_embedding_table, pos_encoding):
    B, T = idx.shape
    V, D = token_embedding_table.shape
    G = T // 2
    SEG_PER_SEQ = 2
    SEQ_PER_CHUNK = 2
    NSEG = SEG_PER_SEQ * SEQ_PER_CHUNK

    info = plsc.get_sparse_core_info()
    NC, NS = info.num_cores, info.num_subcores
    NW = NC * NS
    n_chunks = B // SEQ_PER_CHUNK
    chunks_per_w = n_chunks // NW
    segs_per_w = chunks_per_w * NSEG

    idx2 = idx.reshape(B * SEG_PER_SEQ, G)
    pe2 = pos_encoding.reshape(SEG_PER_SEQ, G, D)

    mesh = plsc.VectorSubcoreMesh(core_axis_name="c", subcore_axis_name="s")

    @functools.partial(
        pl.kernel,
        mesh=mesh,
        compiler_params=pltpu.CompilerParams(use_tc_tiling_on_sc=False),
        out_type=jax.ShapeDtypeStruct((n_chunks, NSEG, G, D), jnp.float32),
        scratch_types=[
            pltpu.VMEM((segs_per_w, G), jnp.int32),
            pltpu.VMEM((2, NSEG, G, D), jnp.float32),
            pltpu.VMEM((SEG_PER_SEQ, G, D), jnp.float32),
            pltpu.SemaphoreType.DMA,
            pltpu.SemaphoreType.DMA,
            pltpu.SemaphoreType.DMA,
            pltpu.SemaphoreType.DMA,
        ],
    )
    def run(idx_hbm, table_hbm, pe_hbm, out_hbm, idx_all, rows, pe_v, g0, g1, o0, o1):
        wid = lax.axis_index("s") * NC + lax.axis_index("c")
        base_c = wid * chunks_per_w
        pltpu.sync_copy(pe_hbm, pe_v)
        pltpu.sync_copy(idx_hbm.at[pl.ds(wid * segs_per_w, segs_per_w)], idx_all)
        gsem = (g0, g1)
        osem = (o0, o1)

        def fire_gathers(t, s):
            for j in range(NSEG):
                pltpu.async_copy(
                    table_hbm.at[idx_all.at[t * NSEG + j]], rows.at[s].at[j], gsem[s]
                )

        def wait_gathers(s):
            pltpu.make_async_copy(out_hbm.at[0], rows.at[s], gsem[s]).wait()

        def wait_out(s):
            pltpu.make_async_copy(out_hbm.at[0], rows.at[s], osem[s]).wait()

        def add_pe(s):
            def body(r, carry):
                for j in range(NSEG):
                    for u in range(D // 16):
                        sl = pl.ds(u * 16, 16)
                        rows[s, j, r, sl] = rows[s, j, r, sl] + pe_v[j % SEG_PER_SEQ, r, sl]
                return carry

            lax.fori_loop(0, G, body, 0)

        def fire_out(t, s):
            pltpu.async_copy(rows.at[s], out_hbm.at[base_c + t], osem[s])

        def step(t, s, first):
            o = 1 - s
            if not first:
                wait_out(o)
            fire_gathers(t + 1, o)
            wait_gathers(s)
            add_pe(s)
            fire_out(t, s)

        fire_gathers(0, 0)
        step(0, 0, first=True)

        def pair(p, carry):
            t1 = 2 * p + 1
            step(t1, 1, first=False)
            step(t1 + 1, 0, first=False)
            return carry

        lax.fori_loop(0, (chunks_per_w - 2) // 2, pair, 0)

        wait_gathers(1)
        add_pe(1)
        fire_out(chunks_per_w - 1, 1)
        wait_out(0)
        wait_out(1)

    out = run(idx2, token_embedding_table, pe2)
    return out.reshape(B, T, D)
